# baseline (speedup 1.0000x reference)
import functools

import jax
import jax.numpy as jnp
from jax import lax
from jax.experimental import pallas as pl
from jax.experimental.pallas import tpu as pltpu
from jax.experimental.pallas import tpu_sc as plsc

B = 16384
L = 50
D = 64
C = 20
V = 1000000

NW = 32
BPW = B // NW
CHUNK = 8
NCHUNK = BPW // CHUNK
NV = D // 16

VCH = 128
NFULL = V // VCH
VTAIL = V - NFULL * VCH
NCHTOT = NFULL + 1
CPW = (NCHTOT + NW - 1) // NW

_mesh = plsc.VectorSubcoreMesh(core_axis_name="c", subcore_axis_name="s")


@functools.partial(
    pl.kernel,
    mesh=_mesh,
    out_type=jax.ShapeDtypeStruct((V * D,), jnp.float32),
    scratch_types=[
        pltpu.VMEM((D, VCH), jnp.float32),
        pltpu.VMEM((D, VCH), jnp.float32),
        pltpu.VMEM((VCH * D,), jnp.float32),
        pltpu.VMEM((VCH * D,), jnp.float32),
        pltpu.SemaphoreType.DMA,
        pltpu.SemaphoreType.DMA,
        pltpu.SemaphoreType.DMA,
        pltpu.SemaphoreType.DMA,
    ],
    compiler_params=pltpu.CompilerParams(
        use_tc_tiling_on_sc=True, needs_layout_passes=False
    ),
)
def _sc_relayout(tt_ref, out_ref, in0, in1, tr0, tr1, sin0, sin1, sout0, sout1):
    inb = (in0, in1)
    trb = (tr0, tr1)
    sin = (sin0, sin1)
    sout = (sout0, sout1)

    wid = lax.axis_index("s") * 2 + lax.axis_index("c")

    def chunk_vstart(ci):
        return pl.multiple_of(jnp.minimum(ci, NFULL) * VCH, VCH)

    def fire_in(ci, par):
        @pl.when(ci < NCHTOT)
        def _():
            pltpu.async_copy(
                tt_ref.at[:, pl.ds(chunk_vstart(ci), VCH)], inb[par], sin[par]
            )

    def drain_in(ci, par):
        @pl.when(ci < NCHTOT)
        def _():
            pltpu.make_async_copy(
                tt_ref.at[:, pl.ds(0, VCH)], inb[par], sin[par]
            ).wait()

    def fire_out(ci, par):
        @pl.when(ci < NFULL)
        def _():
            pltpu.async_copy(
                trb[par], out_ref.at[pl.ds(ci * VCH * D, VCH * D)], sout[par]
            )

        @pl.when(ci == NFULL)
        def _():
            pltpu.async_copy(
                trb[par].at[pl.ds(0, VTAIL * D)],
                out_ref.at[pl.ds((V - VTAIL) * D, VTAIL * D)],
                sout[par],
            )

    def drain_out(ci, par):
        @pl.when(ci < NFULL)
        def _():
            pltpu.make_async_copy(
                trb[par], out_ref.at[pl.ds(0, VCH * D)], sout[par]
            ).wait()

        @pl.when(ci == NFULL)
        def _():
            pltpu.make_async_copy(
                trb[par].at[pl.ds(0, VTAIL * D)],
                out_ref.at[pl.ds(0, VTAIL * D)],
                sout[par],
            ).wait()

    def transpose(par):
        src = inb[par]
        dst = trb[par]
        rowsel = [lax.iota(jnp.int32, 16) + 16 * j for j in range(NV)]
        for v in range(VCH):
            colsel = jnp.full((16,), v, jnp.int32)
            for j in range(NV):
                x = plsc.load_gather(src, [rowsel[j], colsel])
                dst[pl.ds(v * D + 16 * j, 16)] = x

    fire_in(wid, 0)

    def step(k, par):
        ci = wid + NW * k

        @pl.when(k >= 2)
        def _():
            drain_out(ci - 2 * NW, par)

        @pl.when(ci < NCHTOT)
        def _():
            fire_in(ci + NW, 1 - par)
            drain_in(ci, par)
            transpose(par)
            fire_out(ci, par)

    def two_steps(kj, carry):
        step(kj * 2, 0)
        step(kj * 2 + 1, 1)
        return carry

    lax.fori_loop(0, (CPW + 2) // 2, two_steps, 0)

    for k in (CPW - 1, CPW):
        drain_out(wid + NW * k, k % 2)


@functools.partial(
    pl.kernel,
    mesh=_mesh,
    out_type=jax.ShapeDtypeStruct((B, D), jnp.float32),
    scratch_types=[
        pltpu.VMEM((CHUNK, L), jnp.int32),
        pltpu.VMEM((CHUNK, L), jnp.int32),
        pltpu.VMEM((CHUNK * L, D), jnp.float32),
        pltpu.VMEM((CHUNK * L, D), jnp.float32),
        pltpu.VMEM((CHUNK, D), jnp.float32),
        pltpu.VMEM((CHUNK, D), jnp.float32),
        pltpu.SemaphoreType.DMA,
        pltpu.SemaphoreType.DMA,
        pltpu.SemaphoreType.DMA,
        pltpu.SemaphoreType.DMA,
        pltpu.SemaphoreType.DMA,
        pltpu.SemaphoreType.DMA,
    ],
    compiler_params=pltpu.CompilerParams(use_tc_tiling_on_sc=False),
)
def _sc_embed(text_ref, table_ref, out_ref,
              idx0, idx1, rows0, rows1, mean0, mean1,
              sidx0, sidx1, srows0, srows1, sout0, sout1):
    idx = (idx0, idx1)
    rows = (rows0, rows1)
    mean = (mean0, mean1)
    sidx = (sidx0, sidx1)
    srows = (srows0, srows1)
    sout = (sout0, sout1)

    wid = lax.axis_index("s") * 2 + lax.axis_index("c")
    base = wid * BPW

    def fire_gathers(ci, par):
        for b in range(CHUNK):
            pltpu.async_copy(
                table_ref.at[idx[par].at[b]],
                rows[par].at[pl.ds(b * L, L)],
                srows[par],
            )

    def drain_gathers(par):
        pltpu.make_async_copy(
            table_ref.at[pl.ds(0, CHUNK * L)], rows[par], srows[par]
        ).wait()

    def fire_idx(ci, par):
        bag0 = base + ci * CHUNK
        pltpu.async_copy(
            text_ref.at[pl.ds(bag0, CHUNK), :], idx[par], sidx[par]
        )

    def drain_idx(par):
        pltpu.make_async_copy(
            text_ref.at[pl.ds(0, CHUNK), :], idx[par], sidx[par]
        ).wait()

    def drain_out(par):
        pltpu.make_async_copy(
            mean[par], out_ref.at[pl.ds(0, CHUNK), :], sout[par]
        ).wait()

    fire_idx(0, 0)
    drain_idx(0)
    fire_gathers(0, 0)
    fire_idx(1, 1)

    def step(ci, par):
        @pl.when(ci + 1 < NCHUNK)
        def _():
            drain_idx(1 - par)
            fire_gathers(ci + 1, 1 - par)

        @pl.when(ci >= 2)
        def _():
            drain_out(par)

        drain_gathers(par)

        @pl.when(ci + 2 < NCHUNK)
        def _():
            fire_idx(ci + 2, par)
        r = rows[par]
        for b in range(CHUNK):
            def body(l, accs):
                return tuple(
                    accs[v] + r[b * L + l, pl.ds(v * 16, 16)]
                    for v in range(NV)
                )
            accs = lax.fori_loop(
                0, L, body,
                tuple(jnp.zeros((16,), jnp.float32) for _ in range(NV)),
                unroll=10,
            )
            for v in range(NV):
                mean[par][b, pl.ds(v * 16, 16)] = accs[v] * (1.0 / L)

        bag0 = base + ci * CHUNK
        pltpu.async_copy(
            mean[par], out_ref.at[pl.ds(bag0, CHUNK), :], sout[par]
        )

    def two_steps(cj, carry):
        step(cj * 2, 0)
        step(cj * 2 + 1, 1)
        return carry

    lax.fori_loop(0, NCHUNK // 2, two_steps, 0)
    drain_out(0)
    drain_out(1)


def _fc_body(x_ref, w_ref, b_ref, o_ref):
    o_ref[...] = lax.dot_general(
        x_ref[...], w_ref[...],
        dimension_numbers=(((1,), (1,)), ((), ())),
        preferred_element_type=jnp.float32,
    ) + b_ref[...]


def _fc(x, w, b2d):
    bm = 1024
    return pl.pallas_call(
        _fc_body,
        grid=(B // bm,),
        in_specs=[
            pl.BlockSpec((bm, D), lambda i: (i, 0)),
            pl.BlockSpec((C, D), lambda i: (0, 0)),
            pl.BlockSpec((1, C), lambda i: (0, 0)),
        ],
        out_specs=pl.BlockSpec((bm, C), lambda i: (i, 0)),
        out_shape=jax.ShapeDtypeStruct((B, C), jnp.float32),
    )(x, w, b2d)


def kernel(text, emb_weight, fc_w, fc_b):
    table_lin = _sc_relayout(emb_weight.T)
    pooled = _sc_embed(text.astype(jnp.int32), table_lin.reshape(V, D))
    return _fc(pooled, fc_w, fc_b.reshape(1, C))


# speedup vs baseline: 2.0907x; 2.0907x over previous
import functools

import jax
import jax.numpy as jnp
from jax import lax
from jax.experimental import pallas as pl
from jax.experimental.pallas import tpu as pltpu
from jax.experimental.pallas import tpu_sc as plsc

B = 16384
L = 50
D = 64
C = 20
V = 1000000

NW = 32
BPW = B // NW
CHUNK = 8
NCHUNK = BPW // CHUNK
NV = D // 16

VCH = 128
NFULL = V // VCH
VTAIL = V - NFULL * VCH
NCHTOT = NFULL + 1
CPW = (NCHTOT + NW - 1) // NW

_mesh = plsc.VectorSubcoreMesh(core_axis_name="c", subcore_axis_name="s")


@functools.partial(
    pl.kernel,
    mesh=_mesh,
    out_type=jax.ShapeDtypeStruct((V * D,), jnp.float32),
    scratch_types=[
        pltpu.VMEM((D, VCH), jnp.float32),
        pltpu.VMEM((D, VCH), jnp.float32),
        pltpu.VMEM((VCH * D,), jnp.float32),
        pltpu.VMEM((VCH * D,), jnp.float32),
        pltpu.SemaphoreType.DMA,
        pltpu.SemaphoreType.DMA,
        pltpu.SemaphoreType.DMA,
        pltpu.SemaphoreType.DMA,
    ],
    compiler_params=pltpu.CompilerParams(
        use_tc_tiling_on_sc=True, needs_layout_passes=False
    ),
)
def _sc_relayout(tt_ref, out_ref, in0, in1, tr0, tr1, sin0, sin1, sout0, sout1):
    inb = (in0, in1)
    trb = (tr0, tr1)
    sin = (sin0, sin1)
    sout = (sout0, sout1)

    wid = lax.axis_index("s") * 2 + lax.axis_index("c")

    def chunk_vstart(ci):
        return pl.multiple_of(jnp.minimum(ci, NFULL) * VCH, VCH)

    def fire_in(ci, par):
        @pl.when(ci < NCHTOT)
        def _():
            pltpu.async_copy(
                tt_ref.at[:, pl.ds(chunk_vstart(ci), VCH)], inb[par], sin[par]
            )

    def drain_in(ci, par):
        @pl.when(ci < NCHTOT)
        def _():
            pltpu.make_async_copy(
                tt_ref.at[:, pl.ds(0, VCH)], inb[par], sin[par]
            ).wait()

    def fire_out(ci, par):
        @pl.when(ci < NFULL)
        def _():
            pltpu.async_copy(
                trb[par], out_ref.at[pl.ds(ci * VCH * D, VCH * D)], sout[par]
            )

        @pl.when(ci == NFULL)
        def _():
            pltpu.async_copy(
                trb[par].at[pl.ds(0, VTAIL * D)],
                out_ref.at[pl.ds((V - VTAIL) * D, VTAIL * D)],
                sout[par],
            )

    def drain_out(ci, par):
        @pl.when(ci < NFULL)
        def _():
            pltpu.make_async_copy(
                trb[par], out_ref.at[pl.ds(0, VCH * D)], sout[par]
            ).wait()

        @pl.when(ci == NFULL)
        def _():
            pltpu.make_async_copy(
                trb[par].at[pl.ds(0, VTAIL * D)],
                out_ref.at[pl.ds(0, VTAIL * D)],
                sout[par],
            ).wait()

    def transpose(par):
        src = inb[par]
        dst = trb[par]
        iota = lax.iota(jnp.int32, 16)
        iota_d = iota * D
        shifted = [(iota + s) & 15 for s in range(16)]
        def t_body(t, carry):
            colv = iota + 16 * t
            tbase = t * (16 * D)
            for d0 in range(0, D, 16):
                for s in range(16):
                    row = shifted[s] + d0
                    x = plsc.load_gather(src, [row, colv])
                    plsc.store_scatter(dst, [iota_d + row + tbase], x)
            return carry

        lax.fori_loop(0, VCH // 16, t_body, 0)

    fire_in(wid, 0)

    def step(k, par):
        ci = wid + NW * k

        @pl.when(k >= 2)
        def _():
            drain_out(ci - 2 * NW, par)

        @pl.when(ci < NCHTOT)
        def _():
            fire_in(ci + NW, 1 - par)
            drain_in(ci, par)
            transpose(par)
            fire_out(ci, par)

    def two_steps(kj, carry):
        step(kj * 2, 0)
        step(kj * 2 + 1, 1)
        return carry

    lax.fori_loop(0, (CPW + 2) // 2, two_steps, 0)

    for k in (CPW - 1, CPW):
        drain_out(wid + NW * k, k % 2)


@functools.partial(
    pl.kernel,
    mesh=_mesh,
    out_type=jax.ShapeDtypeStruct((B, D), jnp.float32),
    scratch_types=[
        pltpu.VMEM((CHUNK, L), jnp.int32),
        pltpu.VMEM((CHUNK, L), jnp.int32),
        pltpu.VMEM((CHUNK * L, D), jnp.float32),
        pltpu.VMEM((CHUNK * L, D), jnp.float32),
        pltpu.VMEM((CHUNK, D), jnp.float32),
        pltpu.VMEM((CHUNK, D), jnp.float32),
        pltpu.SemaphoreType.DMA,
        pltpu.SemaphoreType.DMA,
        pltpu.SemaphoreType.DMA,
        pltpu.SemaphoreType.DMA,
        pltpu.SemaphoreType.DMA,
        pltpu.SemaphoreType.DMA,
    ],
    compiler_params=pltpu.CompilerParams(use_tc_tiling_on_sc=False),
)
def _sc_embed(text_ref, table_ref, out_ref,
              idx0, idx1, rows0, rows1, mean0, mean1,
              sidx0, sidx1, srows0, srows1, sout0, sout1):
    idx = (idx0, idx1)
    rows = (rows0, rows1)
    mean = (mean0, mean1)
    sidx = (sidx0, sidx1)
    srows = (srows0, srows1)
    sout = (sout0, sout1)

    wid = lax.axis_index("s") * 2 + lax.axis_index("c")
    base = wid * BPW

    def fire_gathers(ci, par):
        for b in range(CHUNK):
            pltpu.async_copy(
                table_ref.at[idx[par].at[b]],
                rows[par].at[pl.ds(b * L, L)],
                srows[par],
            )

    def drain_gathers(par):
        pltpu.make_async_copy(
            table_ref.at[pl.ds(0, CHUNK * L)], rows[par], srows[par]
        ).wait()

    def fire_idx(ci, par):
        bag0 = base + ci * CHUNK
        pltpu.async_copy(
            text_ref.at[pl.ds(bag0, CHUNK), :], idx[par], sidx[par]
        )

    def drain_idx(par):
        pltpu.make_async_copy(
            text_ref.at[pl.ds(0, CHUNK), :], idx[par], sidx[par]
        ).wait()

    def drain_out(par):
        pltpu.make_async_copy(
            mean[par], out_ref.at[pl.ds(0, CHUNK), :], sout[par]
        ).wait()

    fire_idx(0, 0)
    drain_idx(0)
    fire_gathers(0, 0)
    fire_idx(1, 1)

    def step(ci, par):
        @pl.when(ci + 1 < NCHUNK)
        def _():
            drain_idx(1 - par)
            fire_gathers(ci + 1, 1 - par)

        @pl.when(ci >= 2)
        def _():
            drain_out(par)

        drain_gathers(par)

        @pl.when(ci + 2 < NCHUNK)
        def _():
            fire_idx(ci + 2, par)
        r = rows[par]
        for b in range(CHUNK):
            def body(l, accs):
                return tuple(
                    accs[v] + r[b * L + l, pl.ds(v * 16, 16)]
                    for v in range(NV)
                )
            accs = lax.fori_loop(
                0, L, body,
                tuple(jnp.zeros((16,), jnp.float32) for _ in range(NV)),
                unroll=10,
            )
            for v in range(NV):
                mean[par][b, pl.ds(v * 16, 16)] = accs[v] * (1.0 / L)

        bag0 = base + ci * CHUNK
        pltpu.async_copy(
            mean[par], out_ref.at[pl.ds(bag0, CHUNK), :], sout[par]
        )

    def two_steps(cj, carry):
        step(cj * 2, 0)
        step(cj * 2 + 1, 1)
        return carry

    lax.fori_loop(0, NCHUNK // 2, two_steps, 0)
    drain_out(0)
    drain_out(1)


def _fc_body(x_ref, w_ref, b_ref, o_ref):
    o_ref[...] = lax.dot_general(
        x_ref[...], w_ref[...],
        dimension_numbers=(((1,), (1,)), ((), ())),
        preferred_element_type=jnp.float32,
    ) + b_ref[...]


def _fc(x, w, b2d):
    bm = 1024
    return pl.pallas_call(
        _fc_body,
        grid=(B // bm,),
        in_specs=[
            pl.BlockSpec((bm, D), lambda i: (i, 0)),
            pl.BlockSpec((C, D), lambda i: (0, 0)),
            pl.BlockSpec((1, C), lambda i: (0, 0)),
        ],
        out_specs=pl.BlockSpec((bm, C), lambda i: (i, 0)),
        out_shape=jax.ShapeDtypeStruct((B, C), jnp.float32),
    )(x, w, b2d)


def kernel(text, emb_weight, fc_w, fc_b):
    table_lin = _sc_relayout(emb_weight.T)
    pooled = _sc_embed(text.astype(jnp.int32), table_lin.reshape(V, D))
    return _fc(pooled, fc_w, fc_b.reshape(1, C))
